# baseline (speedup 1.0000x reference)
import jax
import jax.numpy as jnp
from jax import lax
from jax.experimental import pallas as pl
from jax.experimental.pallas import tpu as pltpu
from jax.experimental.pallas import tpu_sc as plsc

_L = 16
_CA = 16


def _sc_body(x_hbm, der2_hbm, out_hbm, x_v, der_v0, der_v1, red_v, out_v,
             sem0, sem1):
    D = x_hbm.shape[1]
    R = 3 * D
    CW = _CA * R
    n_chunks = der2_hbm.shape[1] // CW
    wid = lax.axis_index("c") * 16 + lax.axis_index("s")
    lane = lax.broadcasted_iota(jnp.int32, (_L,), 0)
    lrow = lane * R

    pltpu.sync_copy(x_hbm.at[pl.ds(wid * 2, 2)], x_v)

    bufs = (der_v0, der_v1)
    sems = (sem0, sem1)
    zero = jnp.zeros((_L,), jnp.float32)
    out_vec = zero

    for bl in range(2):
        b = wid * 2 + bl
        for u in range(2):
            pltpu.async_copy(der2_hbm.at[b, pl.ds(u * CW, CW)], bufs[u],
                             sems[u])

        def pair_body(t, sq_acc, bl=bl, b=b):
            for u in range(2):
                c = 2 * t + u
                buf, sem = bufs[u], sems[u]
                pltpu.make_async_copy(
                    der2_hbm.at[b, pl.ds(0, CW)], buf, sem).wait()

                def dloop(i, carry, bl=bl, buf=buf):
                    a0, a1, a2 = carry
                    xv = x_v[bl, pl.ds(i * _L, _L)]
                    d0 = i * _L
                    for m in range(_L):
                        idx = lrow + (3 * (d0 + m))
                        g0 = plsc.load_gather(buf, [idx])
                        g1 = plsc.load_gather(buf, [idx + 1])
                        g2 = plsc.load_gather(buf, [idx + 2])
                        xs = xv[m]
                        a0 = a0 + g0 * xs
                        a1 = a1 + g1 * xs
                        a2 = a2 + g2 * xs
                    return (a0, a1, a2)

                a0, a1, a2 = lax.fori_loop(0, D // _L, dloop,
                                           (zero, zero, zero))
                sq_acc = sq_acc + a0 * a0 + a1 * a1 + a2 * a2

                @pl.when(c + 2 < n_chunks)
                def _():
                    pltpu.async_copy(
                        der2_hbm.at[b, pl.ds((c + 2) * CW, CW)], buf, sem)

            return sq_acc

        sq_acc = lax.fori_loop(0, n_chunks // 2, pair_body, zero)

        for s in (8, 4, 2, 1):
            red_v[...] = sq_acc
            sq_acc = sq_acc + plsc.load_gather(red_v, [(lane + s) % _L])
        out_vec = jnp.where(lane == bl, sq_acc, out_vec)

    out_v[...] = out_vec
    pltpu.sync_copy(out_v, out_hbm.at[wid])


def kernel(x, der_desc_wrt_coord):
    B, A, D, K = der_desc_wrt_coord.shape
    der2 = der_desc_wrt_coord.reshape(B, A * D * K)
    f = pl.kernel(
        _sc_body,
        out_type=jax.ShapeDtypeStruct((32, _L), jnp.float32),
        mesh=plsc.VectorSubcoreMesh(core_axis_name="c", subcore_axis_name="s"),
        compiler_params=pltpu.CompilerParams(needs_layout_passes=False),
        scratch_types=[
            pltpu.VMEM((2, D), jnp.float32),
            pltpu.VMEM((_CA * D * K,), jnp.float32),
            pltpu.VMEM((_CA * D * K,), jnp.float32),
            pltpu.VMEM((_L,), jnp.float32),
            pltpu.VMEM((_L,), jnp.float32),
            pltpu.SemaphoreType.DMA,
            pltpu.SemaphoreType.DMA,
        ],
    )
    out2 = f(x, der2)
    return out2[:, :2].reshape(B)
